# baseline (speedup 1.0000x reference)
import jax
import jax.numpy as jnp
from jax.experimental import pallas as pl

_N_CODES = 1024
_DIM = 64
_ROWS = 2048
_BLK = 256
_GRID = _ROWS // _BLK
_CHUNK = 128
_NCHUNK = _N_CODES // _CHUNK
_COMMIT = 0.25


def _chunk_scores(x, cw_ref, j):
    cwj = cw_ref[pl.ds(j * _CHUNK, _CHUNK), :]
    ccj = jnp.sum(cwj * cwj, axis=1)
    xc = jax.lax.dot_general(x, cwj, (((1,), (1,)), ((), ())),
                             precision=jax.lax.Precision.HIGHEST,
                             preferred_element_type=jnp.float32)
    return ccj[None, :] - 2.0 * xc, cwj


def _vq_block(x_ref, cw_ref, q_ref, idx_ref, psum_ref):
    x = x_ref[...]
    iota_l = jax.lax.broadcasted_iota(jnp.int32, (_BLK, _CHUNK), 1)
    big = jnp.full((_BLK, 1), jnp.inf, jnp.float32)
    bigi = jnp.full((_BLK, 1), _N_CODES, jnp.int32)

    m1, i1 = big, bigi
    for j in range(_NCHUNK):
        sj, _ = _chunk_scores(x, cw_ref, j)
        mj = jnp.min(sj, axis=1, keepdims=True)
        ij = jnp.min(jnp.where(sj == mj, iota_l + j * _CHUNK, _N_CODES),
                     axis=1, keepdims=True)
        upd = mj < m1
        m1 = jnp.where(upd, mj, m1)
        i1 = jnp.where(upd, ij, i1)

    m2, i2 = big, bigi
    for j in range(_NCHUNK):
        sj, _ = _chunk_scores(x, cw_ref, j)
        gcol = iota_l + j * _CHUNK
        sj = jnp.where(gcol == i1, jnp.inf, sj)
        mj = jnp.min(sj, axis=1, keepdims=True)
        ij = jnp.min(jnp.where(sj == mj, gcol, _N_CODES),
                     axis=1, keepdims=True)
        upd = mj < m2
        m2 = jnp.where(upd, mj, m2)
        i2 = jnp.where(upd, ij, i2)

    c1 = jnp.zeros((_BLK, _DIM), jnp.float32)
    c2 = jnp.zeros((_BLK, _DIM), jnp.float32)
    for j in range(_NCHUNK):
        cwj = cw_ref[pl.ds(j * _CHUNK, _CHUNK), :]
        gcol = iota_l + j * _CHUNK
        oh1 = (gcol == i1).astype(jnp.float32)
        oh2 = (gcol == i2).astype(jnp.float32)
        c1 = c1 + jax.lax.dot_general(oh1, cwj, (((1,), (0,)), ((), ())),
                                      precision=jax.lax.Precision.HIGHEST,
                                      preferred_element_type=jnp.float32)
        c2 = c2 + jax.lax.dot_general(oh2, cwj, (((1,), (0,)), ((), ())),
                                      precision=jax.lax.Precision.HIGHEST,
                                      preferred_element_type=jnp.float32)

    d1 = jnp.sum((x - c1) ** 2, axis=1, keepdims=True)
    d2 = jnp.sum((x - c2) ** 2, axis=1, keepdims=True)

    use2 = (d2 < d1) | ((d2 == d1) & (i2 < i1))
    idx = jnp.where(use2, i2, i1)
    q = jnp.where(use2, c2, c1)

    q_st = x + (q - x)
    q_ref[...] = q_st
    idx_ref[0, :, :] = idx.reshape(1, _BLK)
    e = (q_st - x) ** 2
    psum_ref[...] = jnp.sum(e).reshape(1, 1, 1)


def kernel(inputs, codewords):
    in_shape = inputs.shape
    x = inputs.reshape(_ROWS, _DIM)

    q_st, idx, psum = pl.pallas_call(
        _vq_block,
        grid=(_GRID,),
        in_specs=[
            pl.BlockSpec((_BLK, _DIM), lambda i: (i, 0)),
            pl.BlockSpec((_N_CODES, _DIM), lambda i: (0, 0)),
        ],
        out_specs=[
            pl.BlockSpec((_BLK, _DIM), lambda i: (i, 0)),
            pl.BlockSpec((1, 1, _BLK), lambda i: (i, 0, 0)),
            pl.BlockSpec((1, 1, 1), lambda i: (i, 0, 0)),
        ],
        out_shape=[
            jax.ShapeDtypeStruct((_ROWS, _DIM), jnp.float32),
            jax.ShapeDtypeStruct((_GRID, 1, _BLK), jnp.int32),
            jax.ShapeDtypeStruct((_GRID, 1, 1), jnp.float32),
        ],
    )(x, codewords)

    mean_e = jnp.sum(psum) / jnp.float32(_ROWS * _DIM)
    loss = mean_e + _COMMIT * mean_e
    return (q_st.reshape(in_shape),
            idx.reshape(in_shape[:-1]),
            loss)
